# baseline (speedup 1.0000x reference)
import functools

import jax
import jax.numpy as jnp
from jax import lax
from jax.experimental import pallas as pl
from jax.experimental.pallas import tpu as pltpu
from jax.experimental.pallas import tpu_sc as plsc

_LANES = 16
_N_SUBCORES = 16
_CHUNK = 48
_GROUP = 4


def _edge_proj_kernel(ef_ref, w_ref, b_ref, o_ref):
    o_ref[...] = (
        jnp.dot(ef_ref[...], w_ref[...], preferred_element_type=jnp.float32)
        + b_ref[...]
    )


def _edge_proj(ef, w, b2d, blk):
    be, d = ef.shape
    m = w.shape[1]
    return pl.pallas_call(
        _edge_proj_kernel,
        grid=(be // blk,),
        in_specs=[
            pl.BlockSpec((blk, d), lambda i: (i, 0)),
            pl.BlockSpec((d, m), lambda i: (0, 0)),
            pl.BlockSpec((1, m), lambda i: (0, 0)),
        ],
        out_specs=pl.BlockSpec((blk, m), lambda i: (i, 0)),
        out_shape=jax.ShapeDtypeStruct((be, m), jnp.float32),
    )(ef, w, b2d)


def _make_sc_mp(B, N, N_pad, E, M):
    edges_per_tile = E // _N_SUBCORES
    n_pipe = (edges_per_tile // _CHUNK // _GROUP) * _GROUP
    n_groups = n_pipe // _GROUP
    rem = edges_per_tile - n_pipe * _CHUNK
    assert rem % 16 == 0 and rem < _CHUNK
    rows_per_tile = N_pad // _N_SUBCORES
    assert N_pad % (_N_SUBCORES * 8) == 0
    rows_last = N - rows_per_tile * (_N_SUBCORES - 1)
    assert rows_last % 8 == 0 and rows_last > 0
    zrep = rows_per_tile // _CHUNK
    zrem = rows_per_tile - zrep * _CHUNK
    assert zrem % 8 == 0
    mesh = plsc.VectorSubcoreMesh(core_axis_name="c", subcore_axis_name="s")

    @functools.partial(
        pl.kernel,
        out_type=jax.ShapeDtypeStruct((B, N, M), jnp.float32),
        mesh=mesh,
        scratch_types=[
            pltpu.VMEM_SHARED((N_pad, M), jnp.float32),
            [pltpu.VMEM((_CHUNK,), jnp.int32)] * 4,
            [pltpu.VMEM((_CHUNK,), jnp.int32)] * 4,
            pltpu.VMEM((max(rem, 16),), jnp.int32),
            pltpu.VMEM((max(rem, 16),), jnp.int32),
            [pltpu.VMEM((_CHUNK, M), jnp.float32)] * 2,
            [pltpu.VMEM((_CHUNK, M), jnp.float32)] * 2,
            [pltpu.VMEM((_CHUNK, M), jnp.float32)] * 2,
            [pltpu.SemaphoreType.DMA] * 4,
            [pltpu.SemaphoreType.DMA] * 4,
            [pltpu.SemaphoreType.DMA] * 2,
            [pltpu.SemaphoreType.DMA] * 2,
            [pltpu.SemaphoreType.DMA] * 2,
        ],
    )
    def k(hidden_hbm, ep_hbm, src_hbm, tgt_hbm, out_hbm,
          acc, src_v, tgt_v, src_t, tgt_t, neigh_v, epv, msg_v,
          ssem, tsem, gsem, esem, wsem):
        c = lax.axis_index("c")
        s = lax.axis_index("s")
        b = c
        row_off = b * N
        r0 = s * rows_per_tile

        def zrow(r, rc):
            zv = jnp.zeros((_LANES,), jnp.float32)
            for j in range(M // _LANES):
                msg_v[1][r, pl.ds(j * _LANES, _LANES)] = zv
            return rc

        lax.fori_loop(0, _CHUNK, zrow, 0)
        for t in range(zrep):
            pltpu.sync_copy(msg_v[1], acc.at[pl.ds(r0 + t * _CHUNK, _CHUNK)])
        if zrem:
            pltpu.sync_copy(msg_v[1].at[pl.ds(0, zrem)],
                            acc.at[pl.ds(r0 + zrep * _CHUNK, zrem)])
        plsc.subcore_barrier()

        tile_base = s * edges_per_tile

        def issue_idx(ci, s4):
            q = b * E + tile_base + ci * _CHUNK
            pltpu.async_copy(src_hbm.at[pl.ds(q, _CHUNK)], src_v[s4],
                             ssem[s4])
            pltpu.async_copy(tgt_hbm.at[pl.ds(q, _CHUNK)], tgt_v[s4],
                             tsem[s4])

        def wait_idx(s4):
            pltpu.make_async_copy(src_hbm.at[pl.ds(0, _CHUNK)], src_v[s4],
                                  ssem[s4]).wait()
            pltpu.make_async_copy(tgt_hbm.at[pl.ds(0, _CHUNK)], tgt_v[s4],
                                  tsem[s4]).wait()
            for t in range(_CHUNK // _LANES):
                sl = pl.ds(t * _LANES, _LANES)
                src_v[s4][sl] = src_v[s4][sl] + row_off

        def issue_data(ci, s4, p):
            q = b * E + tile_base + ci * _CHUNK
            pltpu.async_copy(hidden_hbm.at[src_v[s4]], neigh_v[p], gsem[p])
            pltpu.async_copy(ep_hbm.at[pl.ds(q, _CHUNK)], epv[p], esem[p])

        def wait_data(s4, p):
            pltpu.make_async_copy(hidden_hbm.at[src_v[s4]], neigh_v[p],
                                  gsem[p]).wait()
            pltpu.make_async_copy(ep_hbm.at[pl.ds(0, _CHUNK)], epv[p],
                                  esem[p]).wait()

        def wait_scatter(s4, p):
            pltpu.make_async_copy(msg_v[p], acc.at[tgt_v[s4]], wsem[p]).wait()

        def compute(p, nrows):
            nb, eb, mb = neigh_v[p], epv[p], msg_v[p]

            def row_body(r, rc):
                for j in range(M // _LANES):
                    sl = pl.ds(j * _LANES, _LANES)
                    mb[r, sl] = jnp.maximum(nb[r, sl] + eb[r, sl], 0.0)
                return rc

            lax.fori_loop(0, nrows, row_body, 0)

        issue_idx(0, 0)
        issue_idx(1, 1)
        wait_idx(0)
        issue_data(0, 0, 0)

        def group_body(g, carry):
            for j in range(_GROUP):
                ci = g * _GROUP + j
                p = j % 2

                @pl.when(ci + 1 < n_pipe)
                def _():
                    wait_idx((j + 1) % 4)
                    issue_data(ci + 1, (j + 1) % 4, (j + 1) % 2)

                wait_data(j, p)

                @pl.when(ci >= 2)
                def _():
                    wait_scatter((j + 2) % 4, p)

                compute(p, _CHUNK)
                pltpu.async_copy(msg_v[p], acc.at[tgt_v[j]], wsem[p], add=True)

                @pl.when(ci + 2 < n_pipe)
                def _():
                    issue_idx(ci + 2, (j + 2) % 4)
            return carry

        lax.fori_loop(0, n_groups, group_body, 0)
        wait_scatter((n_pipe - 2) % 4, (n_pipe - 2) % 2)
        wait_scatter((n_pipe - 1) % 4, (n_pipe - 1) % 2)

        if rem:
            q = tile_base + n_pipe * _CHUNK
            qa = b * E + q
            pltpu.sync_copy(src_hbm.at[pl.ds(qa, rem)], src_t)
            pltpu.sync_copy(tgt_hbm.at[pl.ds(qa, rem)], tgt_t)
            for t in range(rem // _LANES):
                sl = pl.ds(t * _LANES, _LANES)
                src_t[sl] = src_t[sl] + row_off
            nv = neigh_v[1].at[pl.ds(0, rem)]
            ev = epv[1].at[pl.ds(0, rem)]
            pltpu.async_copy(hidden_hbm.at[src_t], nv, gsem[1]).wait()
            pltpu.sync_copy(ep_hbm.at[pl.ds(b * E + q, rem)], ev)
            compute(1, rem)
            pltpu.sync_copy(msg_v[1].at[pl.ds(0, rem)], acc.at[tgt_t], add=True)

        plsc.subcore_barrier()

        @pl.when(s < _N_SUBCORES - 1)
        def _():
            pltpu.sync_copy(acc.at[pl.ds(r0, rows_per_tile)],
                            out_hbm.at[b, pl.ds(r0, rows_per_tile)])

        @pl.when(s == _N_SUBCORES - 1)
        def _():
            q = (_N_SUBCORES - 1) * rows_per_tile
            pltpu.sync_copy(acc.at[pl.ds(q, rows_last)],
                            out_hbm.at[b, pl.ds(q, rows_last)])

    return k


def kernel(hidden, edge_features, edge_sources, edge_targets, W_e, b):
    B, N, H = hidden.shape
    _, E, D_E = edge_features.shape
    M = W_e.shape[1]

    n_pad = ((N + _N_SUBCORES * 8 - 1) // (_N_SUBCORES * 8)) * (_N_SUBCORES * 8)
    src = edge_sources.astype(jnp.int32)
    tgt = edge_targets.astype(jnp.int32)

    blk = next(d for d in range(4096, 7, -8) if (B * E) % d == 0)
    ep = _edge_proj(edge_features.reshape(B * E, D_E), W_e,
                    b.reshape(1, M).astype(jnp.float32), blk=blk)
    k = _make_sc_mp(B, N, n_pad, E, M)
    return k(hidden.reshape(B * N, H), ep,
             src.reshape(B * E), tgt.reshape(B * E))


# speedup vs baseline: 1.0072x; 1.0072x over previous
import functools

import jax
import jax.numpy as jnp
from jax import lax
from jax.experimental import pallas as pl
from jax.experimental.pallas import tpu as pltpu
from jax.experimental.pallas import tpu_sc as plsc

_LANES = 16
_N_SUBCORES = 16
_CHUNK = 64
_GROUP = 4


def _edge_proj_kernel(ef_ref, w_ref, b_ref, o_ref):
    o_ref[...] = (
        jnp.dot(ef_ref[...], w_ref[...], preferred_element_type=jnp.float32)
        + b_ref[...]
    )


def _edge_proj(ef, w, b2d, blk):
    be, d = ef.shape
    m = w.shape[1]
    return pl.pallas_call(
        _edge_proj_kernel,
        grid=(be // blk,),
        in_specs=[
            pl.BlockSpec((blk, d), lambda i: (i, 0)),
            pl.BlockSpec((d, m), lambda i: (0, 0)),
            pl.BlockSpec((1, m), lambda i: (0, 0)),
        ],
        out_specs=pl.BlockSpec((blk, m), lambda i: (i, 0)),
        out_shape=jax.ShapeDtypeStruct((be, m), jnp.float32),
    )(ef, w, b2d)


def _make_sc_mp(B, N, N_pad, E, M):
    edges_per_tile = E // _N_SUBCORES
    n_pipe = (edges_per_tile // _CHUNK // _GROUP) * _GROUP
    n_groups = n_pipe // _GROUP
    rem = edges_per_tile - n_pipe * _CHUNK
    assert rem % 16 == 0 and rem < _CHUNK
    rows_per_tile = N_pad // _N_SUBCORES
    assert N_pad % (_N_SUBCORES * 8) == 0
    rows_last = N - rows_per_tile * (_N_SUBCORES - 1)
    assert rows_last % 8 == 0 and rows_last > 0
    zrep = rows_per_tile // _CHUNK
    zrem = rows_per_tile - zrep * _CHUNK
    assert zrem % 8 == 0
    mesh = plsc.VectorSubcoreMesh(core_axis_name="c", subcore_axis_name="s")

    @functools.partial(
        pl.kernel,
        out_type=jax.ShapeDtypeStruct((B, N, M), jnp.float32),
        mesh=mesh,
        scratch_types=[
            pltpu.VMEM_SHARED((N_pad, M), jnp.float32),
            [pltpu.VMEM((_CHUNK,), jnp.int32)] * 4,
            [pltpu.VMEM((_CHUNK,), jnp.int32)] * 4,
            pltpu.VMEM((max(rem, 16),), jnp.int32),
            pltpu.VMEM((max(rem, 16),), jnp.int32),
            [pltpu.VMEM((_CHUNK, M), jnp.float32)] * 4,
            [pltpu.SemaphoreType.DMA] * 4,
            [pltpu.SemaphoreType.DMA] * 4,
            [pltpu.SemaphoreType.DMA] * 4,
            [pltpu.SemaphoreType.DMA] * 4,
            [pltpu.SemaphoreType.DMA] * 4,
        ],
    )
    def k(hidden_hbm, ep_hbm, src_hbm, tgt_hbm, out_hbm,
          acc, src_v, tgt_v, src_t, tgt_t, msg_v,
          ssem, tsem, gsem, esem, wsem):
        c = lax.axis_index("c")
        s = lax.axis_index("s")
        b = c
        row_off = b * N
        r0 = s * rows_per_tile

        def zrow(r, rc):
            zv = jnp.zeros((_LANES,), jnp.float32)
            for j in range(M // _LANES):
                msg_v[1][r, pl.ds(j * _LANES, _LANES)] = zv
            return rc

        lax.fori_loop(0, _CHUNK, zrow, 0)
        for t in range(zrep):
            pltpu.sync_copy(msg_v[1], acc.at[pl.ds(r0 + t * _CHUNK, _CHUNK)])
        if zrem:
            pltpu.sync_copy(msg_v[1].at[pl.ds(0, zrem)],
                            acc.at[pl.ds(r0 + zrep * _CHUNK, zrem)])
        plsc.subcore_barrier()

        tile_base = s * edges_per_tile

        def issue_idx(ci, s4):
            q = b * E + tile_base + ci * _CHUNK
            pltpu.async_copy(src_hbm.at[pl.ds(q, _CHUNK)], src_v[s4],
                             ssem[s4])
            pltpu.async_copy(tgt_hbm.at[pl.ds(q, _CHUNK)], tgt_v[s4],
                             tsem[s4])

        def wait_idx(s4):
            pltpu.make_async_copy(src_hbm.at[pl.ds(0, _CHUNK)], src_v[s4],
                                  ssem[s4]).wait()
            pltpu.make_async_copy(tgt_hbm.at[pl.ds(0, _CHUNK)], tgt_v[s4],
                                  tsem[s4]).wait()
            for t in range(_CHUNK // _LANES):
                sl = pl.ds(t * _LANES, _LANES)
                src_v[s4][sl] = src_v[s4][sl] + row_off

        def issue_ep(ci, p):
            q = b * E + tile_base + ci * _CHUNK
            pltpu.async_copy(ep_hbm.at[pl.ds(q, _CHUNK)], msg_v[p], esem[p])

        def wait_ep(p):
            pltpu.make_async_copy(ep_hbm.at[pl.ds(0, _CHUNK)], msg_v[p],
                                  esem[p]).wait()

        def issue_gather(s4, p):
            pltpu.async_copy(hidden_hbm.at[src_v[s4]], msg_v[p], gsem[p],
                             add=True)

        def wait_gather(s4, p):
            pltpu.make_async_copy(hidden_hbm.at[src_v[s4]], msg_v[p],
                                  gsem[p]).wait()

        def wait_scatter(s4, p):
            pltpu.make_async_copy(msg_v[p], acc.at[tgt_v[s4]], wsem[p]).wait()

        def compute(p, nrows):
            mb = msg_v[p]

            def row_body(r, rc):
                for j in range(M // _LANES):
                    sl = pl.ds(j * _LANES, _LANES)
                    mb[r, sl] = jnp.maximum(mb[r, sl], 0.0)
                return rc

            lax.fori_loop(0, nrows, row_body, 0)

        issue_idx(0, 0)
        issue_idx(1, 1)
        issue_ep(0, 0)
        issue_ep(1, 1)
        wait_idx(0)
        wait_ep(0)
        issue_gather(0, 0)

        def group_body(g, carry):
            for j in range(_GROUP):
                ci = g * _GROUP + j

                @pl.when(ci + 1 < n_pipe)
                def _():
                    wait_idx((j + 1) % 4)
                    wait_ep((j + 1) % 4)
                    issue_gather((j + 1) % 4, (j + 1) % 4)

                wait_gather(j, j)
                compute(j, _CHUNK)
                pltpu.async_copy(msg_v[j], acc.at[tgt_v[j]], wsem[j], add=True)

                @pl.when((ci >= 2) & (ci + 2 < n_pipe))
                def _():
                    wait_scatter((j + 2) % 4, (j + 2) % 4)

                @pl.when(ci + 2 < n_pipe)
                def _():
                    issue_ep(ci + 2, (j + 2) % 4)
                    issue_idx(ci + 2, (j + 2) % 4)
            return carry

        lax.fori_loop(0, n_groups, group_body, 0)
        for j in range(4):
            wait_scatter(j, j)

        if rem:
            q = tile_base + n_pipe * _CHUNK
            qa = b * E + q
            pltpu.sync_copy(src_hbm.at[pl.ds(qa, rem)], src_t)
            pltpu.sync_copy(tgt_hbm.at[pl.ds(qa, rem)], tgt_t)
            for t in range(rem // _LANES):
                sl = pl.ds(t * _LANES, _LANES)
                src_t[sl] = src_t[sl] + row_off
            mv = msg_v[0].at[pl.ds(0, rem)]
            pltpu.sync_copy(ep_hbm.at[pl.ds(b * E + q, rem)], mv)
            pltpu.async_copy(hidden_hbm.at[src_t], mv, gsem[0],
                             add=True).wait()
            compute(0, rem)
            pltpu.sync_copy(msg_v[0].at[pl.ds(0, rem)], acc.at[tgt_t], add=True)

        plsc.subcore_barrier()

        @pl.when(s < _N_SUBCORES - 1)
        def _():
            pltpu.sync_copy(acc.at[pl.ds(r0, rows_per_tile)],
                            out_hbm.at[b, pl.ds(r0, rows_per_tile)])

        @pl.when(s == _N_SUBCORES - 1)
        def _():
            q = (_N_SUBCORES - 1) * rows_per_tile
            pltpu.sync_copy(acc.at[pl.ds(q, rows_last)],
                            out_hbm.at[b, pl.ds(q, rows_last)])

    return k


def kernel(hidden, edge_features, edge_sources, edge_targets, W_e, b):
    B, N, H = hidden.shape
    _, E, D_E = edge_features.shape
    M = W_e.shape[1]

    n_pad = ((N + _N_SUBCORES * 8 - 1) // (_N_SUBCORES * 8)) * (_N_SUBCORES * 8)
    src = edge_sources.astype(jnp.int32)
    tgt = edge_targets.astype(jnp.int32)

    blk = next(d for d in range(4096, 7, -8) if (B * E) % d == 0)
    ep = _edge_proj(edge_features.reshape(B * E, D_E), W_e,
                    b.reshape(1, M).astype(jnp.float32), blk=blk)
    k = _make_sc_mp(B, N, n_pad, E, M)
    return k(hidden.reshape(B * N, H), ep,
             src.reshape(B * E), tgt.reshape(B * E))


# speedup vs baseline: 1.0568x; 1.0493x over previous
import functools

import jax
import jax.numpy as jnp
from jax import lax
from jax.experimental import pallas as pl
from jax.experimental.pallas import tpu as pltpu
from jax.experimental.pallas import tpu_sc as plsc

_LANES = 16
_N_SUBCORES = 16
_CHUNK = 64
_GROUP = 4


def _edge_proj_kernel(ef_ref, w_ref, b_ref, o_ref):
    o_ref[...] = (
        jnp.dot(ef_ref[...], w_ref[...], preferred_element_type=jnp.float32)
        + b_ref[...]
    )


def _edge_proj(ef, w, b2d, blk, B, E, E_half, half):
    d = ef.shape[1]
    m = w.shape[1]
    nb_half = E_half // blk
    nb_e = E // blk

    def ef_map(i):
        return ((i // nb_half) * nb_e + half * nb_half + i % nb_half, 0)

    return pl.pallas_call(
        _edge_proj_kernel,
        grid=(B * nb_half,),
        in_specs=[
            pl.BlockSpec((blk, d), ef_map),
            pl.BlockSpec((d, m), lambda i: (0, 0)),
            pl.BlockSpec((1, m), lambda i: (0, 0)),
        ],
        out_specs=pl.BlockSpec((blk, m), lambda i: (i, 0)),
        out_shape=jax.ShapeDtypeStruct((B * E_half, m), jnp.float32),
    )(ef, w, b2d)


def _make_sc_mp(B, N, N_pad, E, M, E_half, half, init_from_prev):
    edges_per_tile = E_half // _N_SUBCORES
    n_pipe = (edges_per_tile // _CHUNK // _GROUP) * _GROUP
    n_groups = n_pipe // _GROUP
    rem = edges_per_tile - n_pipe * _CHUNK
    assert rem % 16 == 0 and rem < _CHUNK
    rows_per_tile = N_pad // _N_SUBCORES
    assert N_pad % (_N_SUBCORES * 8) == 0
    rows_last = N - rows_per_tile * (_N_SUBCORES - 1)
    assert rows_last % 8 == 0 and rows_last > 0
    zrep = rows_per_tile // _CHUNK
    zrem = rows_per_tile - zrep * _CHUNK
    assert zrem % 8 == 0
    mesh = plsc.VectorSubcoreMesh(core_axis_name="c", subcore_axis_name="s")

    scratch_types = [
        pltpu.VMEM_SHARED((N_pad, M), jnp.float32),
        [pltpu.VMEM((_CHUNK,), jnp.int32)] * 4,
        [pltpu.VMEM((_CHUNK,), jnp.int32)] * 4,
        pltpu.VMEM((max(rem, 16),), jnp.int32),
        pltpu.VMEM((max(rem, 16),), jnp.int32),
        [pltpu.VMEM((_CHUNK, M), jnp.float32)] * 4,
        [pltpu.SemaphoreType.DMA] * 4,
        [pltpu.SemaphoreType.DMA] * 4,
        [pltpu.SemaphoreType.DMA] * 4,
        [pltpu.SemaphoreType.DMA] * 4,
        [pltpu.SemaphoreType.DMA] * 4,
    ]
    out_type = jax.ShapeDtypeStruct((B, N, M), jnp.float32)

    def body(hidden_hbm, ep_hbm, src_hbm, tgt_hbm, prev_hbm, out_hbm,
             acc, src_v, tgt_v, src_t, tgt_t, msg_v,
             ssem, tsem, gsem, esem, wsem):
        c = lax.axis_index("c")
        s = lax.axis_index("s")
        b = c
        row_off = b * N
        r0 = s * rows_per_tile

        if prev_hbm is None:
            def zrow(r, rc):
                zv = jnp.zeros((_LANES,), jnp.float32)
                for j in range(M // _LANES):
                    msg_v[1][r, pl.ds(j * _LANES, _LANES)] = zv
                return rc

            lax.fori_loop(0, _CHUNK, zrow, 0)
            for t in range(zrep):
                pltpu.sync_copy(msg_v[1],
                                acc.at[pl.ds(r0 + t * _CHUNK, _CHUNK)])
            if zrem:
                pltpu.sync_copy(msg_v[1].at[pl.ds(0, zrem)],
                                acc.at[pl.ds(r0 + zrep * _CHUNK, zrem)])
        else:
            @pl.when(s < _N_SUBCORES - 1)
            def _():
                for t in range(rows_per_tile // _CHUNK):
                    sl = pl.ds(r0 + t * _CHUNK, _CHUNK)
                    pltpu.sync_copy(prev_hbm.at[b, sl], acc.at[sl])

            @pl.when(s == _N_SUBCORES - 1)
            def _():
                q = (_N_SUBCORES - 1) * rows_per_tile
                for t in range(rows_last // _CHUNK):
                    sl = pl.ds(q + t * _CHUNK, _CHUNK)
                    pltpu.sync_copy(prev_hbm.at[b, sl], acc.at[sl])
                lr = rows_last % _CHUNK
                if lr:
                    sl = pl.ds(q + (rows_last // _CHUNK) * _CHUNK, lr)
                    pltpu.sync_copy(prev_hbm.at[b, sl], acc.at[sl])

        plsc.subcore_barrier()

        tile_base = s * edges_per_tile

        def issue_idx(ci, s4):
            q = b * E + half * E_half + tile_base + ci * _CHUNK
            pltpu.async_copy(src_hbm.at[pl.ds(q, _CHUNK)], src_v[s4],
                             ssem[s4])
            pltpu.async_copy(tgt_hbm.at[pl.ds(q, _CHUNK)], tgt_v[s4],
                             tsem[s4])

        def wait_idx(s4):
            pltpu.make_async_copy(src_hbm.at[pl.ds(0, _CHUNK)], src_v[s4],
                                  ssem[s4]).wait()
            pltpu.make_async_copy(tgt_hbm.at[pl.ds(0, _CHUNK)], tgt_v[s4],
                                  tsem[s4]).wait()
            for t in range(_CHUNK // _LANES):
                sl = pl.ds(t * _LANES, _LANES)
                src_v[s4][sl] = src_v[s4][sl] + row_off

        def issue_ep(ci, p):
            q = b * E_half + tile_base + ci * _CHUNK
            pltpu.async_copy(ep_hbm.at[pl.ds(q, _CHUNK)], msg_v[p], esem[p])

        def wait_ep(p):
            pltpu.make_async_copy(ep_hbm.at[pl.ds(0, _CHUNK)], msg_v[p],
                                  esem[p]).wait()

        def issue_gather(s4, p):
            pltpu.async_copy(hidden_hbm.at[src_v[s4]], msg_v[p], gsem[p],
                             add=True)

        def wait_gather(s4, p):
            pltpu.make_async_copy(hidden_hbm.at[src_v[s4]], msg_v[p],
                                  gsem[p]).wait()

        def wait_scatter(s4, p):
            pltpu.make_async_copy(msg_v[p], acc.at[tgt_v[s4]], wsem[p]).wait()

        def compute(p, nrows):
            mb = msg_v[p]

            def row_body(r, rc):
                for j in range(M // _LANES):
                    sl = pl.ds(j * _LANES, _LANES)
                    mb[r, sl] = jnp.maximum(mb[r, sl], 0.0)
                return rc

            lax.fori_loop(0, nrows, row_body, 0)

        issue_idx(0, 0)
        issue_idx(1, 1)
        issue_ep(0, 0)
        issue_ep(1, 1)
        wait_idx(0)
        wait_ep(0)
        issue_gather(0, 0)

        def group_body(g, carry):
            for j in range(_GROUP):
                ci = g * _GROUP + j

                @pl.when(ci + 1 < n_pipe)
                def _():
                    wait_idx((j + 1) % 4)
                    wait_ep((j + 1) % 4)
                    issue_gather((j + 1) % 4, (j + 1) % 4)

                wait_gather(j, j)
                compute(j, _CHUNK)
                pltpu.async_copy(msg_v[j], acc.at[tgt_v[j]], wsem[j], add=True)

                @pl.when((ci >= 2) & (ci + 2 < n_pipe))
                def _():
                    wait_scatter((j + 2) % 4, (j + 2) % 4)

                @pl.when(ci + 2 < n_pipe)
                def _():
                    issue_ep(ci + 2, (j + 2) % 4)
                    issue_idx(ci + 2, (j + 2) % 4)
            return carry

        lax.fori_loop(0, n_groups, group_body, 0)
        for j in range(4):
            wait_scatter(j, j)

        if rem:
            q = tile_base + n_pipe * _CHUNK
            qa = b * E + half * E_half + q
            pltpu.sync_copy(src_hbm.at[pl.ds(qa, rem)], src_t)
            pltpu.sync_copy(tgt_hbm.at[pl.ds(qa, rem)], tgt_t)
            for t in range(rem // _LANES):
                sl = pl.ds(t * _LANES, _LANES)
                src_t[sl] = src_t[sl] + row_off
            mv = msg_v[0].at[pl.ds(0, rem)]
            pltpu.sync_copy(ep_hbm.at[pl.ds(b * E_half + q, rem)], mv)
            pltpu.async_copy(hidden_hbm.at[src_t], mv, gsem[0],
                             add=True).wait()
            compute(0, rem)
            pltpu.sync_copy(msg_v[0].at[pl.ds(0, rem)], acc.at[tgt_t], add=True)

        plsc.subcore_barrier()

        @pl.when(s < _N_SUBCORES - 1)
        def _():
            pltpu.sync_copy(acc.at[pl.ds(r0, rows_per_tile)],
                            out_hbm.at[b, pl.ds(r0, rows_per_tile)])

        @pl.when(s == _N_SUBCORES - 1)
        def _():
            q = (_N_SUBCORES - 1) * rows_per_tile
            pltpu.sync_copy(acc.at[pl.ds(q, rows_last)],
                            out_hbm.at[b, pl.ds(q, rows_last)])

    if init_from_prev:
        @functools.partial(pl.kernel, out_type=out_type, mesh=mesh,
                           scratch_types=scratch_types)
        def k(hidden_hbm, ep_hbm, src_hbm, tgt_hbm, prev_hbm, out_hbm,
              acc, src_v, tgt_v, src_t, tgt_t, msg_v,
              ssem, tsem, gsem, esem, wsem):
            body(hidden_hbm, ep_hbm, src_hbm, tgt_hbm, prev_hbm, out_hbm,
                 acc, src_v, tgt_v, src_t, tgt_t, msg_v,
                 ssem, tsem, gsem, esem, wsem)
    else:
        @functools.partial(pl.kernel, out_type=out_type, mesh=mesh,
                           scratch_types=scratch_types)
        def k(hidden_hbm, ep_hbm, src_hbm, tgt_hbm, out_hbm,
              acc, src_v, tgt_v, src_t, tgt_t, msg_v,
              ssem, tsem, gsem, esem, wsem):
            body(hidden_hbm, ep_hbm, src_hbm, tgt_hbm, None, out_hbm,
                 acc, src_v, tgt_v, src_t, tgt_t, msg_v,
                 ssem, tsem, gsem, esem, wsem)

    return k


def kernel(hidden, edge_features, edge_sources, edge_targets, W_e, b):
    B, N, H = hidden.shape
    _, E, D_E = edge_features.shape
    M = W_e.shape[1]
    E_half = E // 2

    n_pad = ((N + _N_SUBCORES * 8 - 1) // (_N_SUBCORES * 8)) * (_N_SUBCORES * 8)
    src = edge_sources.astype(jnp.int32).reshape(B * E)
    tgt = edge_targets.astype(jnp.int32).reshape(B * E)
    hidden2d = hidden.reshape(B * N, H)
    ef_flat = edge_features.reshape(B * E, D_E)
    b2d = b.reshape(1, M).astype(jnp.float32)

    blk = next(d for d in range(4096, 7, -8) if E_half % d == 0)
    ep0 = _edge_proj(ef_flat, W_e, b2d, blk, B, E, E_half, half=0)
    ep1 = _edge_proj(ef_flat, W_e, b2d, blk, B, E, E_half, half=1)
    k0 = _make_sc_mp(B, N, n_pad, E, M, E_half, half=0, init_from_prev=False)
    k1 = _make_sc_mp(B, N, n_pad, E, M, E_half, half=1, init_from_prev=True)
    out0 = k0(hidden2d, ep0, src, tgt)
    return k1(hidden2d, ep1, src, tgt, out0)
